# Initial kernel scaffold; baseline (speedup 1.0000x reference)
import jax
import jax.numpy as jnp
from jax.experimental import pallas as pl


def kernel(x, edge_index, W1, b1, W2, b2):
    raise NotImplementedError("write your pallas kernel here")


# speedup vs baseline: 8.7317x; 8.7317x over previous
import functools

import jax
import jax.numpy as jnp
from jax import lax
from jax.experimental import pallas as pl
from jax.experimental.pallas import tpu as pltpu
from jax.experimental.pallas import tpu_sc as plsc

_SC_INFO = plsc.get_sparse_core_info()
_NC = _SC_INFO.num_cores
_NS = _SC_INFO.num_subcores
_NW = _NC * _NS
_CW = 128


def _make_deg_kernel(n_pad, chunks):
    rps = n_pad // _NS
    mesh = plsc.VectorSubcoreMesh(core_axis_name="c", subcore_axis_name="s")

    @functools.partial(
        pl.kernel,
        mesh=mesh,
        out_type=jax.ShapeDtypeStruct((_NC, n_pad), jnp.float32),
        scratch_types=[
            pltpu.VMEM((chunks, _CW), jnp.int32),
            pltpu.VMEM((_CW,), jnp.float32),
            pltpu.VMEM_SHARED((n_pad,), jnp.float32),
        ],
    )
    def deg_kernel(dst_hbm, ones_hbm, out_hbm, dst_v, ones_v, dacc):
        cid = lax.axis_index("c")
        sid = lax.axis_index("s")
        wid = sid * _NC + cid
        pltpu.sync_copy(ones_hbm.at[pl.ds(0, _CW)], ones_v)
        for r in range(rps // _CW):
            pltpu.sync_copy(
                ones_hbm.at[pl.ds(r * _CW, _CW)],
                dacc.at[pl.ds(sid * rps + r * _CW, _CW)],
            )
        pltpu.sync_copy(dst_hbm.at[wid], dst_v)
        plsc.subcore_barrier()

        def body(j, _):
            pltpu.sync_copy(ones_v, dacc.at[dst_v.at[j]], add=True)
            return 0

        lax.fori_loop(0, chunks, body, 0)
        plsc.subcore_barrier()
        pltpu.sync_copy(
            dacc.at[pl.ds(sid * rps, rps)],
            out_hbm.at[cid, pl.ds(sid * rps, rps)],
        )

    return deg_kernel


def _make_agg_kernel(n_pad, chunks, f):
    rps = n_pad // _NS
    mesh = plsc.VectorSubcoreMesh(core_axis_name="c", subcore_axis_name="s")

    @functools.partial(
        pl.kernel,
        mesh=mesh,
        out_type=jax.ShapeDtypeStruct((_NC, n_pad, f), jnp.float32),
        scratch_types=[
            pltpu.VMEM((chunks, _CW), jnp.int32),
            pltpu.VMEM((chunks, _CW), jnp.int32),
            pltpu.VMEM((_CW, f), jnp.float32),
            pltpu.VMEM_SHARED((n_pad, f), jnp.float32),
            pltpu.SemaphoreType.DMA,
        ],
    )
    def agg_kernel(g_hbm, src_hbm, dst_hbm, out_hbm, src_v, dst_v, rows_v,
                   acc, sem):
        cid = lax.axis_index("c")
        sid = lax.axis_index("s")
        wid = sid * _NC + cid
        pltpu.sync_copy(
            g_hbm.at[pl.ds(sid * rps, rps)],
            acc.at[pl.ds(sid * rps, rps)],
        )
        pltpu.sync_copy(src_hbm.at[wid], src_v)
        pltpu.sync_copy(dst_hbm.at[wid], dst_v)
        plsc.subcore_barrier()

        def body(j, _):
            pltpu.async_copy(g_hbm.at[src_v.at[j]], rows_v, sem).wait()
            pltpu.sync_copy(rows_v, acc.at[dst_v.at[j]], add=True)
            return 0

        lax.fori_loop(0, chunks, body, 0)
        plsc.subcore_barrier()
        pltpu.sync_copy(
            acc.at[pl.ds(sid * rps, rps)],
            out_hbm.at[cid, pl.ds(sid * rps, rps)],
        )

    return agg_kernel


def _mm_scale(xp, w, degp, blk):
    n_pad, fin = xp.shape
    fout = w.shape[1]

    def body(x_ref, w_ref, d_ref, o_ref):
        deg = d_ref[0, :] + d_ref[1, :] - 1.0
        dinv = lax.rsqrt(deg)
        h = jnp.dot(x_ref[...], w_ref[...], preferred_element_type=jnp.float32)
        o_ref[...] = h * dinv[:, None]

    return pl.pallas_call(
        body,
        grid=(n_pad // blk,),
        in_specs=[
            pl.BlockSpec((blk, fin), lambda i: (i, 0)),
            pl.BlockSpec((fin, fout), lambda i: (0, 0)),
            pl.BlockSpec((_NC, blk), lambda i: (0, i)),
        ],
        out_specs=pl.BlockSpec((blk, fout), lambda i: (i, 0)),
        out_shape=jax.ShapeDtypeStruct((n_pad, fout), jnp.float32),
    )(xp, w, degp)


def _combine_mid(p, g1, b1, degp, blk):
    n_pad, f = g1.shape

    def body(p_ref, g_ref, b_ref, d_ref, o_ref):
        deg = d_ref[0, :] + d_ref[1, :] - 1.0
        dinv = lax.rsqrt(deg)[:, None]
        s = p_ref[0] + p_ref[1] - g_ref[...]
        z = jnp.maximum(s * dinv + b_ref[...], 0.0)
        o_ref[...] = z * dinv

    return pl.pallas_call(
        body,
        grid=(n_pad // blk,),
        in_specs=[
            pl.BlockSpec((_NC, blk, f), lambda i: (0, i, 0)),
            pl.BlockSpec((blk, f), lambda i: (i, 0)),
            pl.BlockSpec((1, f), lambda i: (0, 0)),
            pl.BlockSpec((_NC, blk), lambda i: (0, i)),
        ],
        out_specs=pl.BlockSpec((blk, f), lambda i: (i, 0)),
        out_shape=jax.ShapeDtypeStruct((n_pad, f), jnp.float32),
    )(p, g1, b1, degp)


def _combine_mm(q, g2, w2, b2, degp, blk):
    n_pad, f = g2.shape
    fout = w2.shape[1]

    def body(q_ref, g_ref, w_ref, b_ref, d_ref, o_ref):
        deg = d_ref[0, :] + d_ref[1, :] - 1.0
        dinv = lax.rsqrt(deg)[:, None]
        a = (q_ref[0] + q_ref[1] - g_ref[...]) * dinv
        o_ref[...] = (
            jnp.dot(a, w_ref[...], preferred_element_type=jnp.float32)
            + b_ref[...]
        )

    return pl.pallas_call(
        body,
        grid=(n_pad // blk,),
        in_specs=[
            pl.BlockSpec((_NC, blk, f), lambda i: (0, i, 0)),
            pl.BlockSpec((blk, f), lambda i: (i, 0)),
            pl.BlockSpec((f, fout), lambda i: (0, 0)),
            pl.BlockSpec((1, fout), lambda i: (0, 0)),
            pl.BlockSpec((_NC, blk), lambda i: (0, i)),
        ],
        out_specs=pl.BlockSpec((blk, fout), lambda i: (i, 0)),
        out_shape=jax.ShapeDtypeStruct((n_pad, fout), jnp.float32),
    )(q, g2, w2, b2, degp)


def kernel(x, edge_index, W1, b1, W2, b2):
    n, fin = x.shape
    e = edge_index.shape[1]
    f = W1.shape[1]

    blk = 1024
    n_pad = (n // blk + 1) * blk
    e_pad = -(-e // (_NW * _CW)) * (_NW * _CW)
    chunks = e_pad // (_NW * _CW)

    src = edge_index[0].astype(jnp.int32)
    dst = edge_index[1].astype(jnp.int32)
    src = jnp.concatenate([src, jnp.zeros((e_pad - e,), jnp.int32)])
    dst = jnp.concatenate([dst, jnp.full((e_pad - e,), n, jnp.int32)])
    srcr = src.reshape(_NW, chunks, _CW)
    dstr = dst.reshape(_NW, chunks, _CW)

    xp = jnp.pad(x, ((0, n_pad - n), (0, 0)))
    ones = jnp.ones((n_pad // _NS,), jnp.float32)
    b1r = b1.reshape(1, f)
    b2r = b2.reshape(1, fin)

    degp = _make_deg_kernel(n_pad, chunks)(dstr, ones)
    g1 = _mm_scale(xp, W1, degp, blk)
    agg = _make_agg_kernel(n_pad, chunks, f)
    p = agg(g1, srcr, dstr)
    g2 = _combine_mid(p, g1, b1r, degp, blk)
    q = agg(g2, srcr, dstr)
    outp = _combine_mm(q, g2, W2, b2r, degp, blk)
    return outp[:n]
